# baseline (speedup 1.0000x reference)
import functools

import jax
import jax.numpy as jnp
from jax import lax
from jax.experimental import pallas as pl
from jax.experimental.pallas import tpu as pltpu
from jax.experimental.pallas import tpu_sc as plsc

NUM_SELECTS = 8
NOISE_EPS = 0.01
BLW = 0.01
_NEG_BIG = -3.0e38
_INV_SQRT2 = 0.7071067811865476


def _stage1_body(x_ref, wc_ref, w2_ref, noise_t_ref,
                 logits_out, lg_out, nc_out):
    e, blk = noise_t_ref.shape
    mm = jax.lax.dot_general(
        wc_ref[...], x_ref[...], (((1,), (1,)), ((), ())),
        preferred_element_type=jnp.float32,
        precision=jax.lax.Precision.DEFAULT)
    h = jnp.tanh(mm[:e, :])
    noise_mm = mm[e:, :]
    logits_gate = jax.lax.dot_general(
        w2_ref[...], h, (((1,), (0,)), ((), ())),
        preferred_element_type=jnp.float32,
        precision=jax.lax.Precision.DEFAULT)
    noise_control = (jnp.maximum(noise_mm, 0.0)
                     + jnp.log1p(jnp.exp(-jnp.abs(noise_mm))) + NOISE_EPS)
    logits = logits_gate + noise_t_ref[...] * noise_control
    logits_out[...] = logits
    lg_out[...] = logits_gate
    nc_out[...] = noise_control


def _ndtr_sc(z):
    y = z * _INV_SQRT2
    ay = jnp.abs(y)
    t = 1.0 / (1.0 + 0.3275911 * ay)
    poly = t * (0.254829592 + t * (-0.284496736 + t * (
        1.421413741 + t * (-1.453152027 + t * 1.061405429))))
    erf_abs = 1.0 - poly * jnp.exp(-ay * ay)
    erf = jnp.where(y < 0.0, -erf_abs, erf_abs)
    return 0.5 * (1.0 + erf)


def _make_sc_stage(e, b, ns):
    nw = 32
    chunk = b // nw
    lanes = 16
    groups = chunk // lanes
    mesh = plsc.VectorSubcoreMesh(core_axis_name="c", subcore_axis_name="s")

    @functools.partial(
        pl.kernel, mesh=mesh,
        out_type=[
            jax.ShapeDtypeStruct((ns, b), jnp.int32),
            jax.ShapeDtypeStruct((ns, b), jnp.float32),
            jax.ShapeDtypeStruct((nw, e, lanes), jnp.float32),
            jax.ShapeDtypeStruct((nw, e, lanes), jnp.float32),
        ],
        scratch_types=[
            pltpu.VMEM((e, chunk), jnp.float32),
            pltpu.VMEM((e, chunk), jnp.float32),
            pltpu.VMEM((e, chunk), jnp.float32),
            pltpu.VMEM((ns, chunk), jnp.int32),
            pltpu.VMEM((ns, chunk), jnp.float32),
            pltpu.VMEM((e, lanes), jnp.float32),
            pltpu.VMEM((e, lanes), jnp.float32),
        ],
    )
    def sc_stage(logits_hbm, lg_hbm, nc_hbm,
                 idx_hbm, sc_hbm, imp_hbm, load_hbm,
                 lv, gv, nv, idxv, scv, impacc, loadacc):
        wid = lax.axis_index("s") * 2 + lax.axis_index("c")
        base = wid * chunk
        pltpu.sync_copy(logits_hbm.at[:, pl.ds(base, chunk)], lv)
        pltpu.sync_copy(lg_hbm.at[:, pl.ds(base, chunk)], gv)
        pltpu.sync_copy(nc_hbm.at[:, pl.ds(base, chunk)], nv)

        zf = jnp.zeros((lanes,), jnp.float32)
        for ei in range(e):
            impacc[ei] = zf
            loadacc[ei] = zf

        def group(g, carry):
            goff = g * lanes
            t = [jnp.full((lanes,), _NEG_BIG, jnp.float32)
                 for _ in range(ns + 1)]
            ti = [jnp.zeros((lanes,), jnp.int32) for _ in range(ns + 1)]
            for ei in range(e):
                v = lv[ei, pl.ds(goff, lanes)]
                vi = jnp.full((lanes,), ei, jnp.int32)
                for j in range(ns + 1):
                    gt = v > t[j]
                    t[j], v = (jnp.where(gt, v, t[j]),
                               jnp.where(gt, t[j], v))
                    ti[j], vi = (jnp.where(gt, vi, ti[j]),
                                 jnp.where(gt, ti[j], vi))
            m0 = t[0]
            es = [jnp.exp(t[j] - m0) for j in range(ns)]
            den = functools.reduce(jnp.add, es)
            rden = 1.0 / den
            for j in range(ns):
                scv[j, pl.ds(goff, lanes)] = es[j] * rden
                idxv[j, pl.ds(goff, lanes)] = ti[j]
            t_in = t[ns]
            t_out = t[ns - 1]
            for ei in range(e):
                lg = gv[ei, pl.ds(goff, lanes)]
                nc = nv[ei, pl.ds(goff, lanes)]
                lt = lv[ei, pl.ds(goff, lanes)]
                ln = lt - lg
                thr = jnp.where(ln > t_in, t_in, t_out)
                prob = _ndtr_sc((lg - thr) / nc)
                loadacc[ei] = loadacc[ei] + prob
                w = jnp.where(lt >= t_out, jnp.exp(lt - m0) * rden, 0.0)
                impacc[ei] = impacc[ei] + w
            return carry

        lax.fori_loop(0, groups, group, 0)

        pltpu.sync_copy(idxv, idx_hbm.at[:, pl.ds(base, chunk)])
        pltpu.sync_copy(scv, sc_hbm.at[:, pl.ds(base, chunk)])
        pltpu.sync_copy(impacc, imp_hbm.at[wid])
        pltpu.sync_copy(loadacc, load_hbm.at[wid])

    return sc_stage


def _cv_squared(v):
    return jnp.var(v, ddof=1) / (jnp.mean(v) ** 2 + 1e-10)


def kernel(x, W1, W2, Wn, noise):
    b, d = x.shape
    e = W1.shape[0]
    ns = NUM_SELECTS
    wc = jnp.concatenate([W1, Wn], axis=0)
    noise_t = noise.T
    blk = min(2048, b)
    grid = (b // blk,)

    logits_t, lg_t, nc_t = pl.pallas_call(
        _stage1_body,
        grid=grid,
        in_specs=[
            pl.BlockSpec((blk, d), lambda i: (i, 0)),
            pl.BlockSpec((2 * e, d), lambda i: (0, 0)),
            pl.BlockSpec((e, e), lambda i: (0, 0)),
            pl.BlockSpec((e, blk), lambda i: (0, i)),
        ],
        out_specs=(
            pl.BlockSpec((e, blk), lambda i: (0, i)),
            pl.BlockSpec((e, blk), lambda i: (0, i)),
            pl.BlockSpec((e, blk), lambda i: (0, i)),
        ),
        out_shape=(
            jax.ShapeDtypeStruct((e, b), jnp.float32),
            jax.ShapeDtypeStruct((e, b), jnp.float32),
            jax.ShapeDtypeStruct((e, b), jnp.float32),
        ),
    )(x, wc, W2, noise_t)

    sc_stage = _make_sc_stage(e, b, ns)
    idx_t, scores_t, imp_part, load_part = sc_stage(logits_t, lg_t, nc_t)

    importance = imp_part.sum(axis=(0, 2))
    load = load_part.sum(axis=(0, 2))
    balance_loss = (_cv_squared(importance) + _cv_squared(load)) * BLW
    return (idx_t.T, scores_t.T, balance_loss, load, importance)


# speedup vs baseline: 4.8683x; 4.8683x over previous
import functools

import jax
import jax.numpy as jnp
from jax.experimental import pallas as pl
from jax.experimental.pallas import tpu as pltpu

NUM_SELECTS = 8
NOISE_EPS = 0.01
BLW = 0.01
_NEG_BIG = -3.0e38
_INV_SQRT2 = 0.7071067811865476


def _ndtr(z):
    return 0.5 * (1.0 + jax.lax.erf(z * _INV_SQRT2))


def _router_body(x_ref, wc_ref, w2_ref, noise_t_ref,
                 idx_out, scores_out, loss_out, load_out, imp_out):
    i = pl.program_id(0)
    n = pl.num_programs(0)
    e, blk = noise_t_ref.shape

    mm = jax.lax.dot_general(
        wc_ref[...], x_ref[...], (((1,), (1,)), ((), ())),
        preferred_element_type=jnp.float32,
        precision=jax.lax.Precision.DEFAULT)
    h = jnp.tanh(mm[:e, :])
    noise_mm = mm[e:, :]
    logits_gate = jax.lax.dot_general(
        w2_ref[...], h, (((1,), (0,)), ((), ())),
        preferred_element_type=jnp.float32,
        precision=jax.lax.Precision.DEFAULT)
    noise_control = (jnp.maximum(noise_mm, 0.0)
                     + jnp.log1p(jnp.exp(-jnp.abs(noise_mm))) + NOISE_EPS)
    logits_noise = noise_t_ref[...] * noise_control
    logits = logits_gate + logits_noise

    rf = ((e - 1) - jax.lax.broadcasted_iota(jnp.int32, (e, blk), 0)
          ).astype(jnp.float32)
    work = logits
    sel_mask = jnp.zeros((e, blk), jnp.bool_)
    top_vals = []
    top_rmaxs = []
    for k in range(NUM_SELECTS + 1):
        m = jnp.max(work, axis=0, keepdims=True)
        top_vals.append(m)
        if k < NUM_SELECTS:
            rsel = jnp.where(work == m, rf, -1.0)
            rmax = jnp.max(rsel, axis=0, keepdims=True)
            hit = rf == rmax
            top_rmaxs.append(rmax)
            sel_mask = sel_mask | hit
            work = jnp.where(hit, _NEG_BIG, work)

    maxv = top_vals[0]
    exps = [jnp.exp(v - maxv) for v in top_vals[:NUM_SELECTS]]
    denom = functools.reduce(jnp.add, exps)
    scores_out[...] = jnp.concatenate(exps, axis=0) / denom
    idx_out[...] = (jnp.int32(e - 1)
                    - jnp.concatenate(top_rmaxs, axis=0).astype(jnp.int32))

    pe = jnp.where(sel_mask, jnp.exp(logits - maxv), 0.0)
    imp_blk = jnp.sum(pe / denom, axis=1, keepdims=True)

    t_in = top_vals[NUM_SELECTS]
    t_out = top_vals[NUM_SELECTS - 1]
    is_in = logits_noise > t_in
    thr = jnp.where(is_in, t_in, t_out)
    prob = _ndtr((logits_gate - thr) / noise_control)
    load_blk = jnp.sum(prob, axis=1, keepdims=True)

    @pl.when(i == 0)
    def _init():
        imp_out[...] = jnp.zeros_like(imp_out)
        load_out[...] = jnp.zeros_like(load_out)

    imp_out[...] += imp_blk
    load_out[...] += load_blk

    @pl.when(i == n - 1)
    def _finish():
        ef = jnp.float32(e)
        def cv2(v):
            mu = jnp.sum(v) / ef
            var = jnp.sum((v - mu) ** 2) / (ef - 1.0)
            return var / (mu * mu + 1e-10)
        loss = (cv2(imp_out[...]) + cv2(load_out[...])) * BLW
        loss_out[...] = jnp.broadcast_to(loss, (1, 1))


def kernel(x, W1, W2, Wn, noise):
    b, d = x.shape
    e = W1.shape[0]
    ns = NUM_SELECTS
    wc = jnp.concatenate([W1, Wn], axis=0)
    noise_t = noise.T
    blk = min(2048, b)
    grid = (b // blk,)

    out_shapes = (
        jax.ShapeDtypeStruct((ns, b), jnp.int32),
        jax.ShapeDtypeStruct((ns, b), jnp.float32),
        jax.ShapeDtypeStruct((1, 1), jnp.float32),
        jax.ShapeDtypeStruct((e, 1), jnp.float32),
        jax.ShapeDtypeStruct((e, 1), jnp.float32),
    )
    in_specs = [
        pl.BlockSpec((blk, d), lambda i: (i, 0)),
        pl.BlockSpec((2 * e, d), lambda i: (0, 0)),
        pl.BlockSpec((e, e), lambda i: (0, 0)),
        pl.BlockSpec((e, blk), lambda i: (0, i)),
    ]
    out_specs = (
        pl.BlockSpec((ns, blk), lambda i: (0, i)),
        pl.BlockSpec((ns, blk), lambda i: (0, i)),
        pl.BlockSpec((1, 1), lambda i: (0, 0)),
        pl.BlockSpec((e, 1), lambda i: (0, 0)),
        pl.BlockSpec((e, 1), lambda i: (0, 0)),
    )

    idx_t, scores_t, loss, load, imp = pl.pallas_call(
        _router_body,
        grid=grid,
        in_specs=in_specs,
        out_specs=out_specs,
        out_shape=out_shapes,
    )(x, wc, W2, noise_t)
    return (idx_t.T, scores_t.T, loss.reshape(()),
            load.reshape(e), imp.reshape(e))
